# initial kernel scaffold (unmeasured)
import jax
import jax.numpy as jnp
from jax import lax
from jax.experimental import pallas as pl
from jax.experimental.pallas import tpu as pltpu


def kernel(
    x,
):
    def body(*refs):
        pass

    out_shape = jax.ShapeDtypeStruct(..., jnp.float32)
    return pl.pallas_call(body, out_shape=out_shape)(...)


# baseline (device time: 13453 ns/iter reference)
import jax
import jax.numpy as jnp
from jax import lax
from jax.experimental import pallas as pl
from jax.experimental.pallas import tpu as pltpu


def kernel(x):
    _, m, n = x.shape

    def body(x_ref, out_ref, comm_ref, sum_ref, send_sems, recv_sems):
        my_x = lax.axis_index("x")
        my_y = lax.axis_index("y")
        x_nbr = (1 - my_x, my_y)
        y_nbr = (my_x, 1 - my_y)

        barrier_sem = pltpu.get_barrier_semaphore()
        for nbr in (x_nbr, y_nbr):
            pl.semaphore_signal(
                barrier_sem, inc=1,
                device_id=nbr, device_id_type=pl.DeviceIdType.MESH,
            )
        pl.semaphore_wait(barrier_sem, 2)

        rdma_x = pltpu.make_async_remote_copy(
            src_ref=x_ref.at[0],
            dst_ref=comm_ref.at[0],
            send_sem=send_sems.at[0],
            recv_sem=recv_sems.at[0],
            device_id=x_nbr,
            device_id_type=pl.DeviceIdType.MESH,
        )
        rdma_x.start()
        rdma_x.wait()
        sum_ref[...] = x_ref[0] + comm_ref[0]

        rdma_y = pltpu.make_async_remote_copy(
            src_ref=sum_ref,
            dst_ref=comm_ref.at[1],
            send_sem=send_sems.at[1],
            recv_sem=recv_sems.at[1],
            device_id=y_nbr,
            device_id_type=pl.DeviceIdType.MESH,
        )
        rdma_y.start()
        out_ref[:, pl.ds(my_y * n, n)] = sum_ref[...]
        rdma_y.wait()
        out_ref[:, pl.ds((1 - my_y) * n, n)] = comm_ref[1]

    return pl.pallas_call(
        body,
        out_shape=jax.ShapeDtypeStruct((m, 2 * n), jnp.float32),
        in_specs=[pl.BlockSpec(memory_space=pltpu.VMEM)],
        out_specs=pl.BlockSpec(memory_space=pltpu.VMEM),
        scratch_shapes=[
            pltpu.VMEM((2, m, n), jnp.float32),
            pltpu.VMEM((m, n), jnp.float32),
            pltpu.SemaphoreType.DMA((2,)),
            pltpu.SemaphoreType.DMA((2,)),
        ],
        compiler_params=pltpu.CompilerParams(collective_id=0),
    )(x)


# device time: 11462 ns/iter; 1.1737x vs baseline; 1.1737x over previous
import jax
import jax.numpy as jnp
from jax import lax
from jax.experimental import pallas as pl
from jax.experimental.pallas import tpu as pltpu

S = 4


def kernel(x):
    _, m, n = x.shape
    ms = m // S

    def body(x_ref, out_ref, commx_ref, commy_ref, send_sems, recv_sems):
        my_x = lax.axis_index("x")
        my_y = lax.axis_index("y")
        x_nbr = (1 - my_x, my_y)
        y_nbr = (my_x, 1 - my_y)

        barrier_sem = pltpu.get_barrier_semaphore()
        for nbr in (x_nbr, y_nbr):
            pl.semaphore_signal(
                barrier_sem, inc=1,
                device_id=nbr, device_id_type=pl.DeviceIdType.MESH,
            )
        pl.semaphore_wait(barrier_sem, 2)

        rdmas_x = []
        for k in range(S):
            r = pltpu.make_async_remote_copy(
                src_ref=x_ref.at[0, pl.ds(k * ms, ms)],
                dst_ref=commx_ref.at[k],
                send_sem=send_sems.at[0, k],
                recv_sem=recv_sems.at[0, k],
                device_id=x_nbr,
                device_id_type=pl.DeviceIdType.MESH,
            )
            r.start()
            rdmas_x.append(r)

        rdmas_y = []
        for k in range(S):
            rdmas_x[k].wait_recv()
            rows = pl.ds(k * ms, ms)
            out_ref[rows, pl.ds(my_y * n, n)] = (
                x_ref[0, rows, :] + commx_ref[k]
            )
            r = pltpu.make_async_remote_copy(
                src_ref=out_ref.at[rows, pl.ds(my_y * n, n)],
                dst_ref=commy_ref.at[k],
                send_sem=send_sems.at[1, k],
                recv_sem=recv_sems.at[1, k],
                device_id=y_nbr,
                device_id_type=pl.DeviceIdType.MESH,
            )
            r.start()
            rdmas_y.append(r)

        for k in range(S):
            rdmas_y[k].wait_recv()
            out_ref[pl.ds(k * ms, ms), pl.ds((1 - my_y) * n, n)] = commy_ref[k]
        for k in range(S):
            rdmas_x[k].wait_send()
            rdmas_y[k].wait_send()

    return pl.pallas_call(
        body,
        out_shape=jax.ShapeDtypeStruct((m, 2 * n), jnp.float32),
        in_specs=[pl.BlockSpec(memory_space=pltpu.VMEM)],
        out_specs=pl.BlockSpec(memory_space=pltpu.VMEM),
        scratch_shapes=[
            pltpu.VMEM((S, ms, n), jnp.float32),
            pltpu.VMEM((S, ms, n), jnp.float32),
            pltpu.SemaphoreType.DMA((2, S)),
            pltpu.SemaphoreType.DMA((2, S)),
        ],
        compiler_params=pltpu.CompilerParams(collective_id=0),
    )(x)


# device time: 11440 ns/iter; 1.1760x vs baseline; 1.0019x over previous
import jax
import jax.numpy as jnp
from jax import lax
from jax.experimental import pallas as pl
from jax.experimental.pallas import tpu as pltpu

S = 4


def kernel(x):
    _, m, n = x.shape
    ms = m // S

    def body(x_ref, out_ref, commx_ref, send_sems, recv_sems):
        my_x = lax.axis_index("x")
        my_y = lax.axis_index("y")
        x_nbr = (1 - my_x, my_y)
        y_nbr = (my_x, 1 - my_y)

        barrier_sem = pltpu.get_barrier_semaphore()
        for nbr in (x_nbr, y_nbr):
            pl.semaphore_signal(
                barrier_sem, inc=1,
                device_id=nbr, device_id_type=pl.DeviceIdType.MESH,
            )
        pl.semaphore_wait(barrier_sem, 2)

        rdmas_x = []
        for k in range(S):
            r = pltpu.make_async_remote_copy(
                src_ref=x_ref.at[0, pl.ds(k * ms, ms)],
                dst_ref=commx_ref.at[k],
                send_sem=send_sems.at[0, k],
                recv_sem=recv_sems.at[0, k],
                device_id=x_nbr,
                device_id_type=pl.DeviceIdType.MESH,
            )
            r.start()
            rdmas_x.append(r)

        rdmas_y = []
        for k in range(S):
            rdmas_x[k].wait_recv()
            rows = pl.ds(k * ms, ms)
            col = pl.ds(my_y * n, n)
            out_ref[rows, col] = x_ref[0, rows, :] + commx_ref[k]
            r = pltpu.make_async_remote_copy(
                src_ref=out_ref.at[rows, col],
                dst_ref=out_ref.at[rows, col],
                send_sem=send_sems.at[1, k],
                recv_sem=recv_sems.at[1, k],
                device_id=y_nbr,
                device_id_type=pl.DeviceIdType.MESH,
            )
            r.start()
            rdmas_y.append(r)

        for k in range(S):
            rdmas_y[k].wait_recv()
        for k in range(S):
            rdmas_x[k].wait_send()
            rdmas_y[k].wait_send()

    return pl.pallas_call(
        body,
        out_shape=jax.ShapeDtypeStruct((m, 2 * n), jnp.float32),
        in_specs=[pl.BlockSpec(memory_space=pltpu.VMEM)],
        out_specs=pl.BlockSpec(memory_space=pltpu.VMEM),
        scratch_shapes=[
            pltpu.VMEM((S, ms, n), jnp.float32),
            pltpu.SemaphoreType.DMA((2, S)),
            pltpu.SemaphoreType.DMA((2, S)),
        ],
        compiler_params=pltpu.CompilerParams(collective_id=0),
    )(x)


# device time: 11226 ns/iter; 1.1984x vs baseline; 1.0191x over previous
import jax
import jax.numpy as jnp
from jax import lax
from jax.experimental import pallas as pl
from jax.experimental.pallas import tpu as pltpu

S = 8


def kernel(x):
    _, m, n = x.shape
    ms = m // S

    def body(x_ref, out_ref, commx_ref, send_sems, recv_sems):
        my_x = lax.axis_index("x")
        my_y = lax.axis_index("y")
        x_nbr = (1 - my_x, my_y)
        y_nbr = (my_x, 1 - my_y)

        barrier_sem = pltpu.get_barrier_semaphore()
        for nbr in (x_nbr, y_nbr):
            pl.semaphore_signal(
                barrier_sem, inc=1,
                device_id=nbr, device_id_type=pl.DeviceIdType.MESH,
            )
        pl.semaphore_wait(barrier_sem, 2)

        rdmas_x = []
        for k in range(S):
            r = pltpu.make_async_remote_copy(
                src_ref=x_ref.at[0, pl.ds(k * ms, ms)],
                dst_ref=commx_ref.at[k],
                send_sem=send_sems.at[0, k],
                recv_sem=recv_sems.at[0, k],
                device_id=x_nbr,
                device_id_type=pl.DeviceIdType.MESH,
            )
            r.start()
            rdmas_x.append(r)

        rdmas_y = []
        for k in range(S):
            rdmas_x[k].wait_recv()
            for yy in (0, 1):
                @pl.when(my_y == yy)
                def _(k=k, yy=yy):
                    out_ref[pl.ds(k * ms, ms), pl.ds(yy * n, n)] = (
                        x_ref[0, pl.ds(k * ms, ms), :] + commx_ref[k]
                    )

            rows = pl.ds(k * ms, ms)
            col = pl.ds(my_y * n, n)
            r = pltpu.make_async_remote_copy(
                src_ref=out_ref.at[rows, col],
                dst_ref=out_ref.at[rows, col],
                send_sem=send_sems.at[1, k],
                recv_sem=recv_sems.at[1, k],
                device_id=y_nbr,
                device_id_type=pl.DeviceIdType.MESH,
            )
            r.start()
            rdmas_y.append(r)

        for k in range(S):
            rdmas_y[k].wait_recv()
        for k in range(S):
            rdmas_x[k].wait_send()
            rdmas_y[k].wait_send()

    return pl.pallas_call(
        body,
        out_shape=jax.ShapeDtypeStruct((m, 2 * n), jnp.float32),
        in_specs=[pl.BlockSpec(memory_space=pltpu.VMEM)],
        out_specs=pl.BlockSpec(memory_space=pltpu.VMEM),
        scratch_shapes=[
            pltpu.VMEM((S, ms, n), jnp.float32),
            pltpu.SemaphoreType.DMA((2, S)),
            pltpu.SemaphoreType.DMA((2, S)),
        ],
        compiler_params=pltpu.CompilerParams(collective_id=0),
    )(x)
